# Initial kernel scaffold; baseline (speedup 1.0000x reference)
import jax
import jax.numpy as jnp
from jax.experimental import pallas as pl


def kernel(users, items, edge_index, embed_weight):
    raise NotImplementedError("write your pallas kernel here")


# speedup vs baseline: 7.7247x; 7.7247x over previous
import functools

import jax
import jax.numpy as jnp
from jax import lax
from jax.experimental import pallas as pl
from jax.experimental.pallas import tpu as pltpu
from jax.experimental.pallas import tpu_sc as plsc

_NUM_USERS = 5000
_N = 10000
_D = 128
_B = 128
_NPAD = 10240
_PAD_NODE = _N
_E = 320000
_NW = 32
_CHUNK = 128
_NCHUNK = 80
_EPW = _NCHUNK * _CHUNK
_EPAD = _EPW * _NW
_ROWS_PER_TILE = _NPAD // 16
_ZR = 40

_mesh = plsc.VectorSubcoreMesh(core_axis_name="c", subcore_axis_name="s")
_sc_params = pltpu.CompilerParams(needs_layout_passes=False)


@functools.partial(
    pl.kernel,
    mesh=_mesh,
    out_type=[
        jax.ShapeDtypeStruct((_NW, _NPAD), jnp.float32),
        jax.ShapeDtypeStruct((_NW, _NPAD), jnp.float32),
    ],
    scratch_types=[
        pltpu.VMEM((_EPW,), jnp.int32),
        pltpu.VMEM((_EPW,), jnp.int32),
        pltpu.VMEM((_B,), jnp.int32),
        pltpu.VMEM((_NPAD,), jnp.float32),
        pltpu.VMEM((_NPAD,), jnp.float32),
        pltpu.VMEM((_NPAD,), jnp.float32),
    ],
    compiler_params=_sc_params,
)
def _sc_scalar(row_hbm, col_hbm, users_hbm, zeros_hbm,
               deg_out, s_out, row_v, col_v, users_v, uc_v, deg_v, s_v):
    wid = lax.axis_index("s") * 2 + lax.axis_index("c")
    base = wid * _EPW
    pltpu.sync_copy(row_hbm.at[pl.ds(base, _EPW)], row_v)
    pltpu.sync_copy(col_hbm.at[pl.ds(base, _EPW)], col_v)
    pltpu.sync_copy(users_hbm, users_v)
    pltpu.sync_copy(zeros_hbm, uc_v)
    pltpu.sync_copy(zeros_hbm, deg_v)
    pltpu.sync_copy(zeros_hbm, s_v)
    ones = jnp.ones((16,), jnp.float32)

    def users_body(k, carry):
        u16 = users_v[pl.ds(k * 16, 16)]
        plsc.addupdate_scatter(uc_v, [u16], ones)
        return carry

    lax.fori_loop(0, _B // 16, users_body, 0)

    def edge_body(i, carry):
        r16 = row_v[pl.ds(i * 16, 16)]
        c16 = col_v[pl.ds(i * 16, 16)]
        plsc.addupdate_scatter(deg_v, [r16], ones)
        cnt = plsc.load_gather(uc_v, [r16])
        plsc.addupdate_scatter(s_v, [c16], cnt)
        return carry

    lax.fori_loop(0, _EPW // 16, edge_body, 0)
    pltpu.sync_copy(deg_v, deg_out.at[wid])
    pltpu.sync_copy(s_v, s_out.at[wid])


@functools.partial(
    pl.kernel,
    mesh=_mesh,
    out_type=jax.ShapeDtypeStruct((2 * _NPAD, _D), jnp.float32),
    scratch_types=[
        pltpu.VMEM((_NCHUNK, _CHUNK), jnp.int32),
        pltpu.VMEM((_NCHUNK, _CHUNK), jnp.int32),
        pltpu.VMEM((_CHUNK, _D), jnp.float32),
        pltpu.VMEM((_ZR, _D), jnp.float32),
        pltpu.VMEM_SHARED((_NPAD, _D), jnp.float32),
        pltpu.SemaphoreType.DMA,
    ],
    compiler_params=_sc_params,
)
def _sc_edge(y_hbm, row_hbm, col_hbm, zeros_hbm,
             z_out, row_v, col_v, buf, zbuf, z_sh, sem):
    cid = lax.axis_index("c")
    sid = lax.axis_index("s")
    wid = sid * 2 + cid
    row0 = sid * _ROWS_PER_TILE

    pltpu.sync_copy(zeros_hbm, zbuf)

    def zero_body(k, carry):
        pltpu.sync_copy(zbuf, z_sh.at[pl.ds(row0 + k * _ZR, _ZR)])
        return carry

    lax.fori_loop(0, _ROWS_PER_TILE // _ZR, zero_body, 0)
    plsc.subcore_barrier()

    pltpu.sync_copy(row_hbm.at[wid], row_v)
    pltpu.sync_copy(col_hbm.at[wid], col_v)

    def edge_body(j, carry):
        pltpu.async_copy(y_hbm.at[row_v.at[j]], buf, sem).wait()
        pltpu.sync_copy(buf, z_sh.at[col_v.at[j]], add=True)
        return carry

    lax.fori_loop(0, _NCHUNK, edge_body, 0)
    plsc.subcore_barrier()

    goff = cid * _NPAD + row0

    def drain_body(k, carry):
        pltpu.sync_copy(z_sh.at[pl.ds(row0 + k * _CHUNK, _CHUNK)], buf)
        pltpu.sync_copy(buf, z_out.at[pl.ds(goff + k * _CHUNK, _CHUNK)])
        return carry

    lax.fori_loop(0, _ROWS_PER_TILE // _CHUNK, drain_body, 0)


def _tc_scale(x, s):
    def body(x_ref, s_ref, o_ref):
        o_ref[...] = x_ref[...] * s_ref[...]

    return pl.pallas_call(
        body, out_shape=jax.ShapeDtypeStruct((_NPAD, _D), jnp.float32))(x, s)


def _tc_scale_add(z, s):
    def body(z_ref, s_ref, o_ref):
        o_ref[...] = (z_ref[0] + z_ref[1]) * s_ref[...]

    return pl.pallas_call(
        body, out_shape=jax.ShapeDtypeStruct((_NPAD, _D), jnp.float32))(z, s)


def _tc_combine(x3, s, users2, items2):
    def body(x_ref, s_ref, u_ref, i_ref, o_ref):
        x = x_ref[...]
        sv = s_ref[...]
        node_ids = lax.broadcasted_iota(jnp.int32, (_B, _NPAD), 1)
        oh_u = (node_ids == u_ref[...]).astype(jnp.float32)
        oh_i = (node_ids == (i_ref[...] + _NUM_USERS)).astype(jnp.float32)
        u_emb = jnp.dot(oh_u, x, preferred_element_type=jnp.float32,
                        precision=lax.Precision.HIGHEST)
        i_emb = jnp.dot(oh_i, x, preferred_element_type=jnp.float32,
                        precision=lax.Precision.HIGHEST)
        u_ref_v = jnp.sum(u_emb, axis=0, keepdims=True) * (1.0 / _B)
        t = jnp.sum(x * u_ref_v, axis=1, keepdims=True)
        has_any = jnp.max(sv) > 0.0
        mt = jnp.where(sv > 0.0, t, -3.0e38)
        m_safe = jnp.where(has_any, jnp.max(mt), 0.0)
        g = jnp.exp(jnp.minimum(t, m_safe) - m_safe)
        w = sv * g
        wn = w / jnp.maximum(jnp.sum(w), 1e-12)
        neigh = jnp.sum(wn * x, axis=0, keepdims=True)
        base = jnp.sum(u_emb * i_emb, axis=1)
        extra = jnp.sum(u_emb * neigh, axis=1)
        score = base + jnp.where(has_any, extra, 0.0)
        o_ref[...] = (1.0 / (1.0 + jnp.exp(-score)))[None, :]

    return pl.pallas_call(
        body, out_shape=jax.ShapeDtypeStruct((1, _B), jnp.float32))(
            x3, s, users2, items2)


def kernel(users, items, edge_index, embed_weight):
    row = edge_index[0]
    col = edge_index[1]
    row_p = jnp.full((_EPAD,), _PAD_NODE, jnp.int32).at[:_E].set(row)
    col_p = jnp.full((_EPAD,), _PAD_NODE, jnp.int32).at[:_E].set(col)
    zeros1 = jnp.zeros((_NPAD,), jnp.float32)

    deg_parts, s_parts = _sc_scalar(row_p, col_p, users, zeros1)
    deg = jnp.clip(jnp.sum(deg_parts, axis=0), 1.0, None)
    dinv = lax.rsqrt(deg)[:, None]
    dinv2 = (1.0 / deg)[:, None]
    s = jnp.sum(s_parts, axis=0)[:, None]

    x_pad = jnp.zeros((_NPAD, _D), jnp.float32).at[:_N].set(embed_weight)
    y = _tc_scale(x_pad, dinv)

    row3 = row_p.reshape(_NW, _NCHUNK, _CHUNK)
    col3 = col_p.reshape(_NW, _NCHUNK, _CHUNK)
    zrows = jnp.zeros((_ZR, _D), jnp.float32)
    for layer in range(3):
        zparts = _sc_edge(y, row3, col3, zrows)
        y = _tc_scale_add(zparts.reshape(2, _NPAD, _D),
                          dinv2 if layer < 2 else dinv)

    score = _tc_combine(y, s, users.reshape(_B, 1), items.reshape(_B, 1))
    return score.reshape(_B)


# speedup vs baseline: 8.3912x; 1.0863x over previous
import functools

import jax
import jax.numpy as jnp
from jax import lax
from jax.experimental import pallas as pl
from jax.experimental.pallas import tpu as pltpu
from jax.experimental.pallas import tpu_sc as plsc

_NUM_USERS = 5000
_N = 10000
_D = 128
_B = 128
_NPAD = 10240
_PAD_NODE = _N
_E = 320000
_NW = 32
_CHUNK = 64
_NCHUNK = 160
_EPW = _NCHUNK * _CHUNK
_EPAD = _EPW * _NW
_ROWS_PER_TILE = _NPAD // 16
_NBUF = 5

_mesh = plsc.VectorSubcoreMesh(core_axis_name="c", subcore_axis_name="s")
_sc_params = pltpu.CompilerParams(needs_layout_passes=False)


@functools.partial(
    pl.kernel,
    mesh=_mesh,
    out_type=[
        jax.ShapeDtypeStruct((_NW, _NPAD), jnp.float32),
        jax.ShapeDtypeStruct((_NW, _NPAD), jnp.float32),
    ],
    scratch_types=[
        pltpu.VMEM((_EPW,), jnp.int32),
        pltpu.VMEM((_EPW,), jnp.int32),
        pltpu.VMEM((_B,), jnp.int32),
        pltpu.VMEM((_NPAD,), jnp.float32),
        pltpu.VMEM((_NPAD,), jnp.float32),
        pltpu.VMEM((_NPAD,), jnp.float32),
    ],
    compiler_params=_sc_params,
)
def _sc_scalar(row_hbm, col_hbm, users_hbm, zeros_hbm,
               deg_out, s_out, row_v, col_v, users_v, uc_v, deg_v, s_v):
    wid = lax.axis_index("s") * 2 + lax.axis_index("c")
    base = wid * _EPW
    pltpu.sync_copy(row_hbm.at[pl.ds(base, _EPW)], row_v)
    pltpu.sync_copy(col_hbm.at[pl.ds(base, _EPW)], col_v)
    pltpu.sync_copy(users_hbm, users_v)
    pltpu.sync_copy(zeros_hbm, uc_v)
    pltpu.sync_copy(zeros_hbm, deg_v)
    pltpu.sync_copy(zeros_hbm, s_v)
    ones = jnp.ones((16,), jnp.float32)

    def users_body(k, carry):
        u16 = users_v[pl.ds(k * 16, 16)]
        plsc.addupdate_scatter(uc_v, [u16], ones)
        return carry

    lax.fori_loop(0, _B // 16, users_body, 0)

    def edge_body(i, carry):
        r16 = row_v[pl.ds(i * 16, 16)]
        c16 = col_v[pl.ds(i * 16, 16)]
        plsc.addupdate_scatter(deg_v, [r16], ones)
        cnt = plsc.load_gather(uc_v, [r16])
        plsc.addupdate_scatter(s_v, [c16], cnt)
        return carry

    lax.fori_loop(0, _EPW // 16, edge_body, 0)
    pltpu.sync_copy(deg_v, deg_out.at[wid])
    pltpu.sync_copy(s_v, s_out.at[wid])


@functools.partial(
    pl.kernel,
    mesh=_mesh,
    out_type=jax.ShapeDtypeStruct((2 * _NPAD, _D), jnp.float32),
    scratch_types=[
        pltpu.VMEM((_NBUF, _CHUNK), jnp.int32),
        pltpu.VMEM((_NBUF, _CHUNK), jnp.int32),
        pltpu.VMEM((_NBUF * _CHUNK, _D), jnp.float32),
        pltpu.VMEM_SHARED((_NPAD, _D), jnp.float32),
        pltpu.SemaphoreType.DMA((_NBUF,)),
        pltpu.SemaphoreType.DMA((_NBUF,)),
        pltpu.SemaphoreType.DMA((_NBUF,)),
        pltpu.SemaphoreType.DMA((_NBUF,)),
    ],
    compiler_params=_sc_params,
)
def _sc_edge(y_hbm, row_hbm, col_hbm, zeros_hbm,
             z_out, ridx, cidx, buf, z_sh, gsem, ssem, rsem, csem):
    cid = lax.axis_index("c")
    sid = lax.axis_index("s")
    wid = sid * 2 + cid
    row0 = sid * _ROWS_PER_TILE
    ebase = wid * _EPW
    bufs = [buf.at[pl.ds(b * _CHUNK, _CHUNK)] for b in range(_NBUF)]

    def _idx_fetch(j, b):
        pltpu.async_copy(row_hbm.at[pl.ds(ebase + j * _CHUNK, _CHUNK)],
                         ridx.at[b], rsem.at[b])
        pltpu.async_copy(col_hbm.at[pl.ds(ebase + j * _CHUNK, _CHUNK)],
                         cidx.at[b], csem.at[b])

    def _idx_wait(j, b):
        pltpu.make_async_copy(row_hbm.at[pl.ds(ebase + j * _CHUNK, _CHUNK)],
                              ridx.at[b], rsem.at[b]).wait()
        pltpu.make_async_copy(col_hbm.at[pl.ds(ebase + j * _CHUNK, _CHUNK)],
                              cidx.at[b], csem.at[b]).wait()

    pltpu.sync_copy(zeros_hbm, bufs[0])

    def zero_body(k, carry):
        pltpu.sync_copy(bufs[0], z_sh.at[pl.ds(row0 + k * _CHUNK, _CHUNK)])
        return carry

    lax.fori_loop(0, _ROWS_PER_TILE // _CHUNK, zero_body, 0)
    plsc.subcore_barrier()

    for b in range(_NBUF):
        _idx_fetch(b, b)

    def group_body(i, carry):
        j0 = i * _NBUF
        for b in range(_NBUF):
            _idx_wait(j0 + b, b)
            pltpu.async_copy(y_hbm.at[ridx.at[b]], bufs[b], gsem.at[b])
        for b in range(_NBUF):
            pltpu.make_async_copy(y_hbm.at[ridx.at[b]], bufs[b],
                                  gsem.at[b]).wait()
            pltpu.async_copy(bufs[b], z_sh.at[cidx.at[b]], ssem.at[b],
                             add=True)
        for b in range(_NBUF):
            j = j0 + b

            @pl.when(j + _NBUF < _NCHUNK)
            def _():
                pltpu.make_async_copy(bufs[b], z_sh.at[cidx.at[b]],
                                      ssem.at[b]).wait()
                _idx_fetch(j + _NBUF, b)
        return carry

    lax.fori_loop(0, _NCHUNK // _NBUF, group_body, 0)
    for b in range(_NBUF):
        pltpu.make_async_copy(bufs[b], z_sh.at[cidx.at[b]], ssem.at[b]).wait()
    plsc.subcore_barrier()

    goff = cid * _NPAD + row0

    def drain_body(k, carry):
        pltpu.sync_copy(z_sh.at[pl.ds(row0 + k * _CHUNK, _CHUNK)], bufs[0])
        pltpu.sync_copy(bufs[0], z_out.at[pl.ds(goff + k * _CHUNK, _CHUNK)])
        return carry

    lax.fori_loop(0, _ROWS_PER_TILE // _CHUNK, drain_body, 0)


def _tc_scale(x, s):
    def body(x_ref, s_ref, o_ref):
        o_ref[...] = x_ref[...] * s_ref[...]

    return pl.pallas_call(
        body, out_shape=jax.ShapeDtypeStruct((_NPAD, _D), jnp.float32))(x, s)


def _tc_scale_add(z, s):
    def body(z_ref, s_ref, o_ref):
        o_ref[...] = (z_ref[0] + z_ref[1]) * s_ref[...]

    return pl.pallas_call(
        body, out_shape=jax.ShapeDtypeStruct((_NPAD, _D), jnp.float32))(z, s)


def _tc_combine(x3, s, users2, items2):
    def body(x_ref, s_ref, u_ref, i_ref, o_ref):
        x = x_ref[...]
        sv = s_ref[...]
        node_ids = lax.broadcasted_iota(jnp.int32, (_B, _NPAD), 1)
        oh_u = (node_ids == u_ref[...]).astype(jnp.float32)
        oh_i = (node_ids == (i_ref[...] + _NUM_USERS)).astype(jnp.float32)
        u_emb = jnp.dot(oh_u, x, preferred_element_type=jnp.float32,
                        precision=lax.Precision.HIGHEST)
        i_emb = jnp.dot(oh_i, x, preferred_element_type=jnp.float32,
                        precision=lax.Precision.HIGHEST)
        u_ref_v = jnp.sum(u_emb, axis=0, keepdims=True) * (1.0 / _B)
        t = jnp.sum(x * u_ref_v, axis=1, keepdims=True)
        has_any = jnp.max(sv) > 0.0
        mt = jnp.where(sv > 0.0, t, -3.0e38)
        m_safe = jnp.where(has_any, jnp.max(mt), 0.0)
        g = jnp.exp(jnp.minimum(t, m_safe) - m_safe)
        w = sv * g
        wn = w / jnp.maximum(jnp.sum(w), 1e-12)
        neigh = jnp.sum(wn * x, axis=0, keepdims=True)
        base = jnp.sum(u_emb * i_emb, axis=1)
        extra = jnp.sum(u_emb * neigh, axis=1)
        score = base + jnp.where(has_any, extra, 0.0)
        o_ref[...] = (1.0 / (1.0 + jnp.exp(-score)))[None, :]

    return pl.pallas_call(
        body, out_shape=jax.ShapeDtypeStruct((1, _B), jnp.float32))(
            x3, s, users2, items2)


def kernel(users, items, edge_index, embed_weight):
    row = edge_index[0]
    col = edge_index[1]
    row_p = jnp.full((_EPAD,), _PAD_NODE, jnp.int32).at[:_E].set(row)
    col_p = jnp.full((_EPAD,), _PAD_NODE, jnp.int32).at[:_E].set(col)
    zeros1 = jnp.zeros((_NPAD,), jnp.float32)

    deg_parts, s_parts = _sc_scalar(row_p, col_p, users, zeros1)
    deg = jnp.clip(jnp.sum(deg_parts, axis=0), 1.0, None)
    dinv = lax.rsqrt(deg)[:, None]
    dinv2 = (1.0 / deg)[:, None]
    s = jnp.sum(s_parts, axis=0)[:, None]

    x_pad = jnp.zeros((_NPAD, _D), jnp.float32).at[:_N].set(embed_weight)
    y = _tc_scale(x_pad, dinv)

    zrows = jnp.zeros((_CHUNK, _D), jnp.float32)
    for layer in range(3):
        zparts = _sc_edge(y, row_p, col_p, zrows)
        y = _tc_scale_add(zparts.reshape(2, _NPAD, _D),
                          dinv2 if layer < 2 else dinv)

    score = _tc_combine(y, s, users.reshape(_B, 1), items.reshape(_B, 1))
    return score.reshape(_B)


# speedup vs baseline: 9.0930x; 1.0836x over previous
import functools

import jax
import jax.numpy as jnp
from jax import lax
from jax.experimental import pallas as pl
from jax.experimental.pallas import tpu as pltpu
from jax.experimental.pallas import tpu_sc as plsc

_NUM_USERS = 5000
_N = 10000
_D = 128
_B = 128
_NPAD = 10240
_PAD_NODE = _N
_E = 320000
_NW = 32
_CHUNK = 64
_NCHUNK = 160
_NCHUNK0 = 240
_NCHUNK1 = 2 * _NCHUNK - _NCHUNK0
_EPW = _NCHUNK * _CHUNK
_EPAD = _EPW * _NW
_ROWS_PER_TILE = _NPAD // 16
_NBUF = 5

_mesh = plsc.VectorSubcoreMesh(core_axis_name="c", subcore_axis_name="s")
_sc_params = pltpu.CompilerParams(needs_layout_passes=False)


@functools.partial(
    pl.kernel,
    mesh=_mesh,
    out_type=[
        jax.ShapeDtypeStruct((_NW, _NPAD), jnp.float32),
        jax.ShapeDtypeStruct((_NW, _NPAD), jnp.float32),
    ],
    scratch_types=[
        pltpu.VMEM((_EPW,), jnp.int32),
        pltpu.VMEM((_EPW,), jnp.int32),
        pltpu.VMEM((_B,), jnp.int32),
        pltpu.VMEM((_NPAD,), jnp.float32),
        pltpu.VMEM((_NPAD,), jnp.float32),
        pltpu.VMEM((_NPAD,), jnp.float32),
    ],
    compiler_params=_sc_params,
)
def _sc_scalar(row_hbm, col_hbm, users_hbm, zeros_hbm,
               deg_out, s_out, row_v, col_v, users_v, uc_v, deg_v, s_v):
    wid = lax.axis_index("s") * 2 + lax.axis_index("c")
    base = wid * _EPW
    pltpu.sync_copy(row_hbm.at[pl.ds(base, _EPW)], row_v)
    pltpu.sync_copy(col_hbm.at[pl.ds(base, _EPW)], col_v)
    pltpu.sync_copy(users_hbm, users_v)
    pltpu.sync_copy(zeros_hbm, uc_v)
    pltpu.sync_copy(zeros_hbm, deg_v)
    pltpu.sync_copy(zeros_hbm, s_v)
    ones = jnp.ones((16,), jnp.float32)

    def users_body(k, carry):
        u16 = users_v[pl.ds(k * 16, 16)]
        plsc.addupdate_scatter(uc_v, [u16], ones)
        return carry

    lax.fori_loop(0, _B // 16, users_body, 0)

    def edge_body(i, carry):
        r16 = row_v[pl.ds(i * 16, 16)]
        c16 = col_v[pl.ds(i * 16, 16)]
        plsc.addupdate_scatter(deg_v, [r16], ones)
        cnt = plsc.load_gather(uc_v, [r16])
        plsc.addupdate_scatter(s_v, [c16], cnt)
        return carry

    lax.fori_loop(0, _EPW // 16, edge_body, 0)
    pltpu.sync_copy(deg_v, deg_out.at[wid])
    pltpu.sync_copy(s_v, s_out.at[wid])


@functools.partial(
    pl.kernel,
    mesh=_mesh,
    out_type=jax.ShapeDtypeStruct((2 * _NPAD, _D), jnp.float32),
    scratch_types=[
        pltpu.VMEM((_NBUF, _CHUNK), jnp.int32),
        pltpu.VMEM((_NBUF, _CHUNK), jnp.int32),
        pltpu.VMEM((_NBUF * _CHUNK, _D), jnp.float32),
        pltpu.VMEM_SHARED((_NPAD, _D), jnp.float32),
        pltpu.SemaphoreType.DMA((_NBUF,)),
        pltpu.SemaphoreType.DMA((_NBUF,)),
        pltpu.SemaphoreType.DMA((_NBUF,)),
        pltpu.SemaphoreType.DMA((_NBUF,)),
    ],
    compiler_params=_sc_params,
)
def _sc_edge(y_hbm, row_hbm, col_hbm, zeros_hbm,
             z_out, ridx, cidx, buf, z_sh, gsem, ssem, rsem, csem):
    cid = lax.axis_index("c")
    sid = lax.axis_index("s")
    row0 = sid * _ROWS_PER_TILE
    nck = jnp.where(cid == 0, _NCHUNK0, _NCHUNK1)
    ebase = jnp.where(cid == 0, sid * (_NCHUNK0 * _CHUNK),
                      16 * (_NCHUNK0 * _CHUNK) + sid * (_NCHUNK1 * _CHUNK))
    bufs = [buf.at[pl.ds(b * _CHUNK, _CHUNK)] for b in range(_NBUF)]

    def _idx_fetch(j, b):
        pltpu.async_copy(row_hbm.at[pl.ds(ebase + j * _CHUNK, _CHUNK)],
                         ridx.at[b], rsem.at[b])
        pltpu.async_copy(col_hbm.at[pl.ds(ebase + j * _CHUNK, _CHUNK)],
                         cidx.at[b], csem.at[b])

    def _idx_wait(j, b):
        pltpu.make_async_copy(row_hbm.at[pl.ds(ebase + j * _CHUNK, _CHUNK)],
                              ridx.at[b], rsem.at[b]).wait()
        pltpu.make_async_copy(col_hbm.at[pl.ds(ebase + j * _CHUNK, _CHUNK)],
                              cidx.at[b], csem.at[b]).wait()

    pltpu.sync_copy(zeros_hbm, bufs[0])

    def zero_body(k, carry):
        pltpu.sync_copy(bufs[0], z_sh.at[pl.ds(row0 + k * _CHUNK, _CHUNK)])
        return carry

    lax.fori_loop(0, _ROWS_PER_TILE // _CHUNK, zero_body, 0)
    plsc.subcore_barrier()

    for b in range(_NBUF):
        _idx_fetch(b, b)

    def group_body(i, carry):
        j0 = i * _NBUF
        for b in range(_NBUF):
            _idx_wait(j0 + b, b)
            pltpu.async_copy(y_hbm.at[ridx.at[b]], bufs[b], gsem.at[b])
        for b in range(_NBUF):
            pltpu.make_async_copy(y_hbm.at[ridx.at[b]], bufs[b],
                                  gsem.at[b]).wait()
            pltpu.async_copy(bufs[b], z_sh.at[cidx.at[b]], ssem.at[b],
                             add=True)
        for b in range(_NBUF):
            j = j0 + b

            @pl.when(j + _NBUF < nck)
            def _():
                pltpu.make_async_copy(bufs[b], z_sh.at[cidx.at[b]],
                                      ssem.at[b]).wait()
                _idx_fetch(j + _NBUF, b)
        return carry

    lax.fori_loop(0, nck // _NBUF, group_body, 0)
    for b in range(_NBUF):
        pltpu.make_async_copy(bufs[b], z_sh.at[cidx.at[b]], ssem.at[b]).wait()
    plsc.subcore_barrier()

    goff = cid * _NPAD + row0

    def drain_body(k, carry):
        pltpu.sync_copy(z_sh.at[pl.ds(row0 + k * _CHUNK, _CHUNK)], bufs[0])
        pltpu.sync_copy(bufs[0], z_out.at[pl.ds(goff + k * _CHUNK, _CHUNK)])
        return carry

    lax.fori_loop(0, _ROWS_PER_TILE // _CHUNK, drain_body, 0)


def _tc_scale(x, s):
    def body(x_ref, s_ref, o_ref):
        o_ref[...] = x_ref[...] * s_ref[...]

    return pl.pallas_call(
        body, out_shape=jax.ShapeDtypeStruct((_NPAD, _D), jnp.float32))(x, s)


def _tc_scale_add(z, s):
    def body(z_ref, s_ref, o_ref):
        o_ref[...] = (z_ref[0] + z_ref[1]) * s_ref[...]

    return pl.pallas_call(
        body, out_shape=jax.ShapeDtypeStruct((_NPAD, _D), jnp.float32))(z, s)


def _tc_combine(x3, s, users2, items2):
    def body(x_ref, s_ref, u_ref, i_ref, o_ref):
        x = x_ref[...]
        sv = s_ref[...]
        node_ids = lax.broadcasted_iota(jnp.int32, (_B, _NPAD), 1)
        oh_u = (node_ids == u_ref[...]).astype(jnp.float32)
        oh_i = (node_ids == (i_ref[...] + _NUM_USERS)).astype(jnp.float32)
        u_emb = jnp.dot(oh_u, x, preferred_element_type=jnp.float32,
                        precision=lax.Precision.HIGHEST)
        i_emb = jnp.dot(oh_i, x, preferred_element_type=jnp.float32,
                        precision=lax.Precision.HIGHEST)
        u_ref_v = jnp.sum(u_emb, axis=0, keepdims=True) * (1.0 / _B)
        t = jnp.sum(x * u_ref_v, axis=1, keepdims=True)
        has_any = jnp.max(sv) > 0.0
        mt = jnp.where(sv > 0.0, t, -3.0e38)
        m_safe = jnp.where(has_any, jnp.max(mt), 0.0)
        g = jnp.exp(jnp.minimum(t, m_safe) - m_safe)
        w = sv * g
        wn = w / jnp.maximum(jnp.sum(w), 1e-12)
        neigh = jnp.sum(wn * x, axis=0, keepdims=True)
        base = jnp.sum(u_emb * i_emb, axis=1)
        extra = jnp.sum(u_emb * neigh, axis=1)
        score = base + jnp.where(has_any, extra, 0.0)
        o_ref[...] = (1.0 / (1.0 + jnp.exp(-score)))[None, :]

    return pl.pallas_call(
        body, out_shape=jax.ShapeDtypeStruct((1, _B), jnp.float32))(
            x3, s, users2, items2)


def kernel(users, items, edge_index, embed_weight):
    row = edge_index[0]
    col = edge_index[1]
    row_p = jnp.full((_EPAD,), _PAD_NODE, jnp.int32).at[:_E].set(row)
    col_p = jnp.full((_EPAD,), _PAD_NODE, jnp.int32).at[:_E].set(col)
    zeros1 = jnp.zeros((_NPAD,), jnp.float32)

    deg_parts, s_parts = _sc_scalar(row_p, col_p, users, zeros1)
    deg = jnp.clip(jnp.sum(deg_parts, axis=0), 1.0, None)
    dinv = lax.rsqrt(deg)[:, None]
    dinv2 = (1.0 / deg)[:, None]
    s = jnp.sum(s_parts, axis=0)[:, None]

    x_pad = jnp.zeros((_NPAD, _D), jnp.float32).at[:_N].set(embed_weight)
    y = _tc_scale(x_pad, dinv)

    zrows = jnp.zeros((_CHUNK, _D), jnp.float32)
    for layer in range(3):
        zparts = _sc_edge(y, row_p, col_p, zrows)
        y = _tc_scale_add(zparts.reshape(2, _NPAD, _D),
                          dinv2 if layer < 2 else dinv)

    score = _tc_combine(y, s, users.reshape(_B, 1), items.reshape(_B, 1))
    return score.reshape(_B)


# speedup vs baseline: 9.1773x; 1.0093x over previous
import functools

import jax
import jax.numpy as jnp
from jax import lax
from jax.experimental import pallas as pl
from jax.experimental.pallas import tpu as pltpu
from jax.experimental.pallas import tpu_sc as plsc

_NUM_USERS = 5000
_N = 10000
_D = 128
_B = 128
_NPAD = 10240
_PAD_NODE = _N
_E = 320000
_NW = 32
_CHUNK = 64
_NCHUNK = 160
_NCHUNK0 = 240
_NCHUNK1 = 2 * _NCHUNK - _NCHUNK0
_EPW = _NCHUNK * _CHUNK
_EPAD = _EPW * _NW
_ROWS_PER_TILE = _NPAD // 16
_NBUF = 5

_mesh = plsc.VectorSubcoreMesh(core_axis_name="c", subcore_axis_name="s")
_sc_params = pltpu.CompilerParams(needs_layout_passes=False)


@functools.partial(
    pl.kernel,
    mesh=_mesh,
    out_type=[
        jax.ShapeDtypeStruct((_NW, _NPAD), jnp.float32),
        jax.ShapeDtypeStruct((_NW, _NPAD), jnp.float32),
    ],
    scratch_types=[
        pltpu.VMEM((_EPW,), jnp.int32),
        pltpu.VMEM((_EPW,), jnp.int32),
        pltpu.VMEM((_B,), jnp.int32),
        pltpu.VMEM((_NPAD,), jnp.float32),
        pltpu.VMEM((_NPAD,), jnp.float32),
        pltpu.VMEM((_NPAD,), jnp.float32),
    ],
    compiler_params=_sc_params,
)
def _sc_scalar(row_hbm, col_hbm, users_hbm, zeros_hbm,
               deg_out, s_out, row_v, col_v, users_v, uc_v, deg_v, s_v):
    wid = lax.axis_index("s") * 2 + lax.axis_index("c")
    base = wid * _EPW
    pltpu.sync_copy(row_hbm.at[pl.ds(base, _EPW)], row_v)
    pltpu.sync_copy(col_hbm.at[pl.ds(base, _EPW)], col_v)
    pltpu.sync_copy(users_hbm, users_v)
    pltpu.sync_copy(zeros_hbm, uc_v)
    pltpu.sync_copy(zeros_hbm, deg_v)
    pltpu.sync_copy(zeros_hbm, s_v)
    ones = jnp.ones((16,), jnp.float32)

    def users_body(k, carry):
        u16 = users_v[pl.ds(k * 16, 16)]
        plsc.addupdate_scatter(uc_v, [u16], ones)
        return carry

    lax.fori_loop(0, _B // 16, users_body, 0)

    def edge_body(i, carry):
        r16 = row_v[pl.ds(i * 16, 16)]
        c16 = col_v[pl.ds(i * 16, 16)]
        plsc.addupdate_scatter(deg_v, [r16], ones)
        cnt = plsc.load_gather(uc_v, [r16])
        plsc.addupdate_scatter(s_v, [c16], cnt)
        return carry

    lax.fori_loop(0, _EPW // 16, edge_body, 0)
    pltpu.sync_copy(deg_v, deg_out.at[wid])
    pltpu.sync_copy(s_v, s_out.at[wid])


@functools.partial(
    pl.kernel,
    mesh=_mesh,
    out_type=jax.ShapeDtypeStruct((2 * _NPAD, _D), jnp.float32),
    scratch_types=[
        pltpu.VMEM((_NBUF, _CHUNK), jnp.int32),
        pltpu.VMEM((_NBUF, _CHUNK), jnp.int32),
        pltpu.VMEM((_NBUF * _CHUNK, _D), jnp.float32),
        pltpu.VMEM_SHARED((_NPAD, _D), jnp.float32),
        pltpu.SemaphoreType.DMA((_NBUF,)),
        pltpu.SemaphoreType.DMA((_NBUF,)),
        pltpu.SemaphoreType.DMA((_NBUF,)),
        pltpu.SemaphoreType.DMA((_NBUF,)),
    ],
    compiler_params=_sc_params,
)
def _sc_edge(y_hbm, row_hbm, col_hbm, zeros_hbm,
             z_out, ridx, cidx, buf, z_sh, gsem, ssem, rsem, csem):
    cid = lax.axis_index("c")
    sid = lax.axis_index("s")
    row0 = sid * _ROWS_PER_TILE
    nck = jnp.where(cid == 0, _NCHUNK0, _NCHUNK1)
    ebase = jnp.where(cid == 0, sid * (_NCHUNK0 * _CHUNK),
                      16 * (_NCHUNK0 * _CHUNK) + sid * (_NCHUNK1 * _CHUNK))
    bufs = [buf.at[pl.ds(b * _CHUNK, _CHUNK)] for b in range(_NBUF)]

    def _idx_fetch(j, b):
        pltpu.async_copy(row_hbm.at[pl.ds(ebase + j * _CHUNK, _CHUNK)],
                         ridx.at[b], rsem.at[b])
        pltpu.async_copy(col_hbm.at[pl.ds(ebase + j * _CHUNK, _CHUNK)],
                         cidx.at[b], csem.at[b])

    def _idx_wait(j, b):
        pltpu.make_async_copy(row_hbm.at[pl.ds(ebase + j * _CHUNK, _CHUNK)],
                              ridx.at[b], rsem.at[b]).wait()
        pltpu.make_async_copy(col_hbm.at[pl.ds(ebase + j * _CHUNK, _CHUNK)],
                              cidx.at[b], csem.at[b]).wait()

    nzc = _ROWS_PER_TILE // _CHUNK
    pltpu.sync_copy(zeros_hbm, bufs[0])
    for k in range(nzc):
        pltpu.async_copy(bufs[0], z_sh.at[pl.ds(row0 + k * _CHUNK, _CHUNK)],
                         gsem.at[k % _NBUF])
    for k in range(nzc):
        pltpu.make_async_copy(bufs[0],
                              z_sh.at[pl.ds(row0 + k * _CHUNK, _CHUNK)],
                              gsem.at[k % _NBUF]).wait()
    plsc.subcore_barrier()

    for b in range(_NBUF):
        _idx_fetch(b, b)

    def group_body(i, carry):
        j0 = i * _NBUF
        for b in range(_NBUF):
            _idx_wait(j0 + b, b)
            pltpu.async_copy(y_hbm.at[ridx.at[b]], bufs[b], gsem.at[b])
        for b in range(_NBUF):
            pltpu.make_async_copy(y_hbm.at[ridx.at[b]], bufs[b],
                                  gsem.at[b]).wait()
            pltpu.async_copy(bufs[b], z_sh.at[cidx.at[b]], ssem.at[b],
                             add=True)
        for b in range(_NBUF):
            j = j0 + b

            @pl.when(j + _NBUF < nck)
            def _():
                pltpu.make_async_copy(bufs[b], z_sh.at[cidx.at[b]],
                                      ssem.at[b]).wait()
                _idx_fetch(j + _NBUF, b)
        return carry

    lax.fori_loop(0, nck // _NBUF, group_body, 0)
    for b in range(_NBUF):
        pltpu.make_async_copy(bufs[b], z_sh.at[cidx.at[b]], ssem.at[b]).wait()
    plsc.subcore_barrier()

    goff = cid * _NPAD + row0

    def _dread(k, b):
        return pltpu.make_async_copy(
            z_sh.at[pl.ds(row0 + k * _CHUNK, _CHUNK)], bufs[b], gsem.at[b])

    def _dwrite(k, b):
        return pltpu.make_async_copy(
            bufs[b], z_out.at[pl.ds(goff + k * _CHUNK, _CHUNK)], ssem.at[b])

    for b in range(_NBUF):
        _dread(b, b).start()
    for g in range(nzc // _NBUF):
        for b in range(_NBUF):
            k = g * _NBUF + b
            _dread(k, b).wait()
            _dwrite(k, b).start()
        for b in range(_NBUF):
            k = g * _NBUF + b
            if k + _NBUF < nzc:
                _dwrite(k, b).wait()
                _dread(k + _NBUF, b).start()
    for b in range(_NBUF):
        _dwrite(nzc - _NBUF + b, b).wait()


def _tc_scale(x, s):
    def body(x_ref, s_ref, o_ref):
        o_ref[...] = x_ref[...] * s_ref[...]

    return pl.pallas_call(
        body, out_shape=jax.ShapeDtypeStruct((_NPAD, _D), jnp.float32))(x, s)


def _tc_scale_add(z, s):
    def body(z_ref, s_ref, o_ref):
        o_ref[...] = (z_ref[0] + z_ref[1]) * s_ref[...]

    return pl.pallas_call(
        body, out_shape=jax.ShapeDtypeStruct((_NPAD, _D), jnp.float32))(z, s)


def _tc_combine(x3, s, users2, items2):
    def body(x_ref, s_ref, u_ref, i_ref, o_ref):
        x = x_ref[...]
        sv = s_ref[...]
        node_ids = lax.broadcasted_iota(jnp.int32, (_B, _NPAD), 1)
        oh_u = (node_ids == u_ref[...]).astype(jnp.float32)
        oh_i = (node_ids == (i_ref[...] + _NUM_USERS)).astype(jnp.float32)
        u_emb = jnp.dot(oh_u, x, preferred_element_type=jnp.float32,
                        precision=lax.Precision.HIGHEST)
        i_emb = jnp.dot(oh_i, x, preferred_element_type=jnp.float32,
                        precision=lax.Precision.HIGHEST)
        u_ref_v = jnp.sum(u_emb, axis=0, keepdims=True) * (1.0 / _B)
        t = jnp.sum(x * u_ref_v, axis=1, keepdims=True)
        has_any = jnp.max(sv) > 0.0
        mt = jnp.where(sv > 0.0, t, -3.0e38)
        m_safe = jnp.where(has_any, jnp.max(mt), 0.0)
        g = jnp.exp(jnp.minimum(t, m_safe) - m_safe)
        w = sv * g
        wn = w / jnp.maximum(jnp.sum(w), 1e-12)
        neigh = jnp.sum(wn * x, axis=0, keepdims=True)
        base = jnp.sum(u_emb * i_emb, axis=1)
        extra = jnp.sum(u_emb * neigh, axis=1)
        score = base + jnp.where(has_any, extra, 0.0)
        o_ref[...] = (1.0 / (1.0 + jnp.exp(-score)))[None, :]

    return pl.pallas_call(
        body, out_shape=jax.ShapeDtypeStruct((1, _B), jnp.float32))(
            x3, s, users2, items2)


def kernel(users, items, edge_index, embed_weight):
    row = edge_index[0]
    col = edge_index[1]
    row_p = jnp.full((_EPAD,), _PAD_NODE, jnp.int32).at[:_E].set(row)
    col_p = jnp.full((_EPAD,), _PAD_NODE, jnp.int32).at[:_E].set(col)
    zeros1 = jnp.zeros((_NPAD,), jnp.float32)

    deg_parts, s_parts = _sc_scalar(row_p, col_p, users, zeros1)
    deg = jnp.clip(jnp.sum(deg_parts, axis=0), 1.0, None)
    dinv = lax.rsqrt(deg)[:, None]
    dinv2 = (1.0 / deg)[:, None]
    s = jnp.sum(s_parts, axis=0)[:, None]

    x_pad = jnp.zeros((_NPAD, _D), jnp.float32).at[:_N].set(embed_weight)
    y = _tc_scale(x_pad, dinv)

    zrows = jnp.zeros((_CHUNK, _D), jnp.float32)
    for layer in range(3):
        zparts = _sc_edge(y, row_p, col_p, zrows)
        y = _tc_scale_add(zparts.reshape(2, _NPAD, _D),
                          dinv2 if layer < 2 else dinv)

    score = _tc_combine(y, s, users.reshape(_B, 1), items.reshape(_B, 1))
    return score.reshape(_B)
